# baseline (speedup 1.0000x reference)
import functools

import jax
import jax.numpy as jnp
from jax import lax
from jax.experimental import pallas as pl
from jax.experimental.pallas import tpu as pltpu
from jax.experimental.pallas import tpu_sc as plsc

_N = 10000
_D = 128
_HD = 64
_NP = 10240
_NS = 16
_NCH = 160
_CH = 128
_PAD_SRC = 10200


def _mm_body(x_ref, w_ref, a_ref, h_ref, s_ref):
    h = jnp.dot(x_ref[...], w_ref[...], preferred_element_type=jnp.float32)
    h_ref[...] = h
    s_ref[...] = jnp.dot(h, a_ref[...], preferred_element_type=jnp.float32)


def _dense_part(x, W, a8):
    return pl.pallas_call(
        _mm_body,
        grid=(10,),
        in_specs=[
            pl.BlockSpec((1000, _D), lambda i: (i, 0)),
            pl.BlockSpec((_D, _D), lambda i: (0, 0)),
            pl.BlockSpec((_D, 8), lambda i: (0, 0)),
        ],
        out_specs=[
            pl.BlockSpec((1000, _D), lambda i: (i, 0)),
            pl.BlockSpec((1000, 8), lambda i: (i, 0)),
        ],
        out_shape=[
            jax.ShapeDtypeStruct((_N, _D), jnp.float32),
            jax.ShapeDtypeStruct((_N, 8), jnp.float32),
        ],
    )(x, W, a8)


def _sc_body(h2_hbm, s1_hbm, s2_hbm, src_hbm, dst_hbm,
             hp_hbm,
             src_v, dst_v, s1_v, s2_v, w0, w1, g0, g1, g2, sb0, sb1, sb2,
             zb_v, accum, rowsum,
             sg0, sg1, sg2, ss0, ss1, ss2, sw0, sw1,
             sd0, sd1, sd2, sd3, sd4, sd5):
    cid = lax.axis_index("c")
    sid = lax.axis_index("s")
    gbufs = (g0, g1, g2)
    sbufs = (sb0, sb1, sb2)
    sgs = (sg0, sg1, sg2)
    sss = (ss0, ss1, ss2)
    wbufs = (w0, w1)
    sws = (sw0, sw1)
    sds = (sd0, sd1, sd2, sd3, sd4, sd5)

    pltpu.sync_copy(src_hbm.at[sid], src_v)
    pltpu.sync_copy(s1_hbm, s1_v)
    pltpu.sync_copy(s2_hbm, s2_v)

    off = jnp.full((16,), cid * _N, jnp.int32)

    def _stream_dst(c):
        pltpu.async_copy(
            dst_hbm.at[cid, sid, c], dst_v.at[c % 6], sds[c % 6])

    def _wait_dst(slot6):
        pltpu.make_async_copy(
            src_hbm.at[sid, 0], dst_v.at[slot6], sds[slot6]).wait()

    def _issue_gather(c):
        _wait_dst(c % 6)
        pltpu.async_copy(
            h2_hbm.at[dst_v.at[c % 6]], gbufs[c % 3], sgs[c % 3])

    zero16 = jnp.zeros((16,), jnp.float32)

    @plsc.parallel_loop(0, _CH, unroll=4)
    def _zrow(i):
        for d in range(_HD // 16):
            sb0[i, pl.ds(d * 16, 16)] = zero16

    @plsc.parallel_loop(0, 40, unroll=4)
    def _zzb(i):
        zb_v[pl.ds(i * 16, 16)] = zero16

    base = sid * 640
    for k in range(5):
        pltpu.sync_copy(sb0, accum.at[pl.ds(base + k * _CH, _CH)])
    pltpu.sync_copy(zb_v, rowsum.at[pl.ds(base, 640)])

    for c in range(6):
        _stream_dst(c)
    for c in range(3):
        _issue_gather(c)
    plsc.subcore_barrier()

    def _process(j, j6, skip_w_wait=False, skip_row_wait=False):
        j3 = j6 % 3
        j2 = j6 % 2
        gbuf, sbuf = gbufs[j3], sbufs[j3]
        wbuf, sem_w = wbufs[j2], sws[j2]
        pltpu.make_async_copy(h2_hbm.at[pl.ds(0, _CH)], gbuf, sgs[j3]).wait()

        if not skip_w_wait:
            pltpu.make_async_copy(
                wbuf, rowsum.at[src_v.at[0]], sem_w).wait()
        for g in range(8):
            srcv = src_v[j, pl.ds(g * 16, 16)]
            dstv = dst_v[j6, pl.ds(g * 16, 16)] - off
            lg = plsc.load_gather(s1_v, [srcv]) + plsc.load_gather(s2_v, [dstv])
            wbuf[pl.ds(g * 16, 16)] = jnp.exp(-jnp.maximum(lg, 0.2 * lg))
        pltpu.async_copy(wbuf, rowsum.at[src_v.at[j]], sem_w, add=True)

        if not skip_row_wait:
            pltpu.make_async_copy(
                sbuf, accum.at[src_v.at[0]], sss[j3]).wait()

        @plsc.parallel_loop(0, _CH, unroll=4)
        def _srow(i):
            wv = plsc.load_gather(wbuf, [jnp.full((16,), i, jnp.int32)])
            for d in range(_HD // 32):
                ab = gbuf[i, pl.ds(d * 32, 32)]
                a, b = plsc.unpack(ab, format=plsc.PackFormat.INTERLEAVED)
                sbuf[i, pl.ds(d * 32, 16)] = a * wv
                sbuf[i, pl.ds(d * 32 + 16, 16)] = b * wv

        pltpu.async_copy(sbuf, accum.at[src_v.at[j]], sss[j3], add=True)

        @pl.when(j + 6 < _NCH)
        def _():
            _stream_dst_dyn(j + 6, j6)
        _issue_gather_dyn(j + 3, (j6 + 3) % 6)

    def _stream_dst_dyn(c, slot6):
        pltpu.async_copy(
            dst_hbm.at[cid, sid, c], dst_v.at[slot6], sds[slot6])

    def _issue_gather_dyn(c, slot6):
        @pl.when(c < _NCH)
        def _():
            _wait_dst(slot6)
            pltpu.async_copy(
                h2_hbm.at[dst_v.at[slot6]], gbufs[slot6 % 3], sgs[slot6 % 3])

    _process(0, 0, skip_w_wait=True, skip_row_wait=True)
    _process(1, 1, skip_w_wait=True, skip_row_wait=True)
    _process(2, 2, skip_row_wait=True)
    _process(3, 3)

    def _outer(t, c):
        j = 4 + 6 * t
        for k in range(6):
            _process(j + k, (4 + k) % 6)
        return c

    lax.fori_loop(0, (_NCH - 4) // 6, _outer, 0)

    pltpu.make_async_copy(w0, rowsum.at[src_v.at[0]], sw0).wait()
    pltpu.make_async_copy(w1, rowsum.at[src_v.at[0]], sw1).wait()
    for c in range(3):
        pltpu.make_async_copy(
            sbufs[c], accum.at[src_v.at[0]], sss[c]).wait()

    plsc.subcore_barrier()
    pltpu.sync_copy(rowsum.at[pl.ds(base, 640)], zb_v)
    for k in range(5):
        pltpu.sync_copy(accum.at[pl.ds(base + k * _CH, _CH)], sb0)

        @plsc.parallel_loop(0, _CH, unroll=2)
        def _nrow(i):
            rsb = plsc.load_gather(
                zb_v, [jnp.full((16,), i + k * _CH, jnp.int32)])
            rinv = 1.0 / (rsb + 1e-16)
            for d in range(_HD // 16):
                x = sb0[i, pl.ds(d * 16, 16)] * rinv
                sb0[i, pl.ds(d * 16, 16)] = jnp.where(
                    x > 0, x, jnp.exp(x) - 1.0)

        pltpu.sync_copy(sb0, hp_hbm.at[cid, pl.ds(base + k * _CH, _CH)])


def _sparse_part(h2b, s1p, s2p, src3, dst4):
    mesh = plsc.VectorSubcoreMesh(core_axis_name="c", subcore_axis_name="s")
    fn = functools.partial(
        pl.kernel,
        mesh=mesh,
        compiler_params=pltpu.CompilerParams(
            needs_layout_passes=False, use_tc_tiling_on_sc=False),
        out_type=jax.ShapeDtypeStruct((2, _NP, _HD), jnp.float32),
        scratch_types=[
            pltpu.VMEM((_NCH, _CH), jnp.int32),
            pltpu.VMEM((6, _CH), jnp.int32),
            pltpu.VMEM((_NP,), jnp.float32),
            pltpu.VMEM((_NP,), jnp.float32),
            pltpu.VMEM((_CH,), jnp.float32),
            pltpu.VMEM((_CH,), jnp.float32),
            pltpu.VMEM((_CH, _HD), jnp.bfloat16),
            pltpu.VMEM((_CH, _HD), jnp.bfloat16),
            pltpu.VMEM((_CH, _HD), jnp.bfloat16),
            pltpu.VMEM((_CH, _HD), jnp.float32),
            pltpu.VMEM((_CH, _HD), jnp.float32),
            pltpu.VMEM((_CH, _HD), jnp.float32),
            pltpu.VMEM((640,), jnp.float32),
            pltpu.VMEM_SHARED((_NP, _HD), jnp.float32),
            pltpu.VMEM_SHARED((_NP,), jnp.float32),
        ] + [pltpu.SemaphoreType.DMA] * 14,
    )(_sc_body)
    return fn(h2b, s1p, s2p, src3, dst4)


def kernel(entity_table, W, a, edge_index):
    a8 = jnp.zeros((_D, 8), jnp.float32)
    a8 = a8.at[:, 0].set(a[0, :_D]).at[:, 1].set(a[0, _D:])
    h, s = _dense_part(entity_table, W, a8)
    h2 = jnp.concatenate([h[:, :_HD], h[:, _HD:]], axis=0)
    perm = []
    for g in range(_HD // 32):
        for i in range(16):
            perm.extend([g * 32 + i, g * 32 + 16 + i])
    h2b = h2.astype(jnp.bfloat16)[:, jnp.array(perm, jnp.int32)]
    s1p = jnp.pad(s[:, 0], (0, _NP - _N))
    s2p = jnp.pad(s[:, 1], (0, _NP - _N))

    e = edge_index.shape[1]
    pad = _NS * _NCH * _CH - e
    src3 = jnp.concatenate(
        [edge_index[0], jnp.full((pad,), _PAD_SRC, jnp.int32)]
    ).reshape(_NS, _NCH, _CH)
    dstp = jnp.concatenate(
        [edge_index[1], jnp.zeros((pad,), jnp.int32)])
    dst4 = jnp.stack([dstp, dstp + _N]).reshape(2, _NS, _NCH, _CH)

    hp = _sparse_part(h2b, s1p, s2p, src3, dst4)
    return jnp.concatenate([hp[0, :_N], hp[1, :_N]], axis=1)


# speedup vs baseline: 1.0013x; 1.0013x over previous
import functools

import jax
import jax.numpy as jnp
from jax import lax
from jax.experimental import pallas as pl
from jax.experimental.pallas import tpu as pltpu
from jax.experimental.pallas import tpu_sc as plsc

_N = 10000
_D = 128
_HD = 64
_NP = 10240
_NS = 16
_NCH = 160
_CH = 128
_PAD_SRC = 10200


def _mm_body(x_ref, w_ref, a_ref, h_ref, s_ref):
    h = jnp.dot(x_ref[...], w_ref[...], preferred_element_type=jnp.float32)
    h_ref[...] = h
    s_ref[...] = jnp.dot(h, a_ref[...], preferred_element_type=jnp.float32)


def _dense_part(x, W, a8):
    return pl.pallas_call(
        _mm_body,
        grid=(10,),
        in_specs=[
            pl.BlockSpec((1000, _D), lambda i: (i, 0)),
            pl.BlockSpec((_D, _D), lambda i: (0, 0)),
            pl.BlockSpec((_D, 8), lambda i: (0, 0)),
        ],
        out_specs=[
            pl.BlockSpec((1000, _D), lambda i: (i, 0)),
            pl.BlockSpec((1000, 8), lambda i: (i, 0)),
        ],
        out_shape=[
            jax.ShapeDtypeStruct((_N, _D), jnp.float32),
            jax.ShapeDtypeStruct((_N, 8), jnp.float32),
        ],
    )(x, W, a8)


def _sc_body(h2_hbm, s1_hbm, s2_hbm, src_hbm, dst_hbm,
             hp_hbm,
             src_v, dst_v, s1_v, s2_v, w0, w1, g0, g1, g2, sb0, sb1, sb2,
             zb_v, accum, rowsum,
             sg0, sg1, sg2, ss0, ss1, ss2, sw0, sw1,
             sd0, sd1, sd2, sd3, sd4, sd5):
    cid = lax.axis_index("c")
    sid = lax.axis_index("s")
    gbufs = (g0, g1, g2)
    sbufs = (sb0, sb1, sb2)
    sgs = (sg0, sg1, sg2)
    sss = (ss0, ss1, ss2)
    wbufs = (w0, w1)
    sws = (sw0, sw1)
    sds = (sd0, sd1, sd2, sd3, sd4, sd5)

    pltpu.sync_copy(src_hbm.at[sid], src_v)
    pltpu.sync_copy(s1_hbm, s1_v)
    pltpu.sync_copy(s2_hbm, s2_v)

    off = jnp.full((16,), cid * _N, jnp.int32)

    def _stream_dst(c):
        pltpu.async_copy(
            dst_hbm.at[cid, sid, c], dst_v.at[c % 6], sds[c % 6])

    def _wait_dst(slot6):
        pltpu.make_async_copy(
            src_hbm.at[sid, 0], dst_v.at[slot6], sds[slot6]).wait()

    def _issue_gather(c):
        _wait_dst(c % 6)
        pltpu.async_copy(
            h2_hbm.at[dst_v.at[c % 6]], gbufs[c % 3], sgs[c % 3])

    zero16 = jnp.zeros((16,), jnp.float32)

    @plsc.parallel_loop(0, _CH, unroll=4)
    def _zrow(i):
        for d in range(_HD // 16):
            sb0[i, pl.ds(d * 16, 16)] = zero16

    @plsc.parallel_loop(0, 40, unroll=4)
    def _zzb(i):
        zb_v[pl.ds(i * 16, 16)] = zero16

    base = sid * 640
    for k in range(5):
        pltpu.sync_copy(sb0, accum.at[pl.ds(base + k * _CH, _CH)])
    pltpu.sync_copy(zb_v, rowsum.at[pl.ds(base, 640)])

    for c in range(6):
        _stream_dst(c)
    for c in range(3):
        _issue_gather(c)
    plsc.subcore_barrier()

    def _process(j, j6, skip_w_wait=False, skip_row_wait=False):
        j3 = j6 % 3
        j2 = j6 % 2
        gbuf, sbuf = gbufs[j3], sbufs[j3]
        wbuf, sem_w = wbufs[j2], sws[j2]
        if not skip_w_wait:
            pltpu.make_async_copy(
                wbuf, rowsum.at[src_v.at[0]], sem_w).wait()
        for g in range(8):
            srcv = src_v[j, pl.ds(g * 16, 16)]
            dstv = dst_v[j6, pl.ds(g * 16, 16)] - off
            lg = plsc.load_gather(s1_v, [srcv]) + plsc.load_gather(s2_v, [dstv])
            wbuf[pl.ds(g * 16, 16)] = jnp.exp(-jnp.maximum(lg, 0.2 * lg))
        pltpu.async_copy(wbuf, rowsum.at[src_v.at[j]], sem_w, add=True)

        pltpu.make_async_copy(h2_hbm.at[pl.ds(0, _CH)], gbuf, sgs[j3]).wait()

        if not skip_row_wait:
            pltpu.make_async_copy(
                sbuf, accum.at[src_v.at[0]], sss[j3]).wait()

        @plsc.parallel_loop(0, _CH, unroll=4)
        def _srow(i):
            wv = plsc.load_gather(wbuf, [jnp.full((16,), i, jnp.int32)])
            for d in range(_HD // 32):
                ab = gbuf[i, pl.ds(d * 32, 32)]
                a, b = plsc.unpack(ab, format=plsc.PackFormat.INTERLEAVED)
                sbuf[i, pl.ds(d * 32, 16)] = a * wv
                sbuf[i, pl.ds(d * 32 + 16, 16)] = b * wv

        pltpu.async_copy(sbuf, accum.at[src_v.at[j]], sss[j3], add=True)

        @pl.when(j + 6 < _NCH)
        def _():
            _stream_dst_dyn(j + 6, j6)
        _issue_gather_dyn(j + 3, (j6 + 3) % 6)

    def _stream_dst_dyn(c, slot6):
        pltpu.async_copy(
            dst_hbm.at[cid, sid, c], dst_v.at[slot6], sds[slot6])

    def _issue_gather_dyn(c, slot6):
        @pl.when(c < _NCH)
        def _():
            _wait_dst(slot6)
            pltpu.async_copy(
                h2_hbm.at[dst_v.at[slot6]], gbufs[slot6 % 3], sgs[slot6 % 3])

    _process(0, 0, skip_w_wait=True, skip_row_wait=True)
    _process(1, 1, skip_w_wait=True, skip_row_wait=True)
    _process(2, 2, skip_row_wait=True)
    _process(3, 3)

    def _outer(t, c):
        j = 4 + 6 * t
        for k in range(6):
            _process(j + k, (4 + k) % 6)
        return c

    lax.fori_loop(0, (_NCH - 4) // 6, _outer, 0)

    pltpu.make_async_copy(w0, rowsum.at[src_v.at[0]], sw0).wait()
    pltpu.make_async_copy(w1, rowsum.at[src_v.at[0]], sw1).wait()
    for c in range(3):
        pltpu.make_async_copy(
            sbufs[c], accum.at[src_v.at[0]], sss[c]).wait()

    plsc.subcore_barrier()
    pltpu.sync_copy(rowsum.at[pl.ds(base, 640)], zb_v)
    for k in range(5):
        pltpu.sync_copy(accum.at[pl.ds(base + k * _CH, _CH)], sb0)

        @plsc.parallel_loop(0, _CH, unroll=2)
        def _nrow(i):
            rsb = plsc.load_gather(
                zb_v, [jnp.full((16,), i + k * _CH, jnp.int32)])
            rinv = 1.0 / (rsb + 1e-16)
            for d in range(_HD // 16):
                x = sb0[i, pl.ds(d * 16, 16)] * rinv
                sb0[i, pl.ds(d * 16, 16)] = jnp.where(
                    x > 0, x, jnp.exp(x) - 1.0)

        pltpu.sync_copy(sb0, hp_hbm.at[cid, pl.ds(base + k * _CH, _CH)])


def _sparse_part(h2b, s1p, s2p, src3, dst4):
    mesh = plsc.VectorSubcoreMesh(core_axis_name="c", subcore_axis_name="s")
    fn = functools.partial(
        pl.kernel,
        mesh=mesh,
        compiler_params=pltpu.CompilerParams(
            needs_layout_passes=False, use_tc_tiling_on_sc=False),
        out_type=jax.ShapeDtypeStruct((2, _NP, _HD), jnp.float32),
        scratch_types=[
            pltpu.VMEM((_NCH, _CH), jnp.int32),
            pltpu.VMEM((6, _CH), jnp.int32),
            pltpu.VMEM((_NP,), jnp.float32),
            pltpu.VMEM((_NP,), jnp.float32),
            pltpu.VMEM((_CH,), jnp.float32),
            pltpu.VMEM((_CH,), jnp.float32),
            pltpu.VMEM((_CH, _HD), jnp.bfloat16),
            pltpu.VMEM((_CH, _HD), jnp.bfloat16),
            pltpu.VMEM((_CH, _HD), jnp.bfloat16),
            pltpu.VMEM((_CH, _HD), jnp.float32),
            pltpu.VMEM((_CH, _HD), jnp.float32),
            pltpu.VMEM((_CH, _HD), jnp.float32),
            pltpu.VMEM((640,), jnp.float32),
            pltpu.VMEM_SHARED((_NP, _HD), jnp.float32),
            pltpu.VMEM_SHARED((_NP,), jnp.float32),
        ] + [pltpu.SemaphoreType.DMA] * 14,
    )(_sc_body)
    return fn(h2b, s1p, s2p, src3, dst4)


def kernel(entity_table, W, a, edge_index):
    a8 = jnp.zeros((_D, 8), jnp.float32)
    a8 = a8.at[:, 0].set(a[0, :_D]).at[:, 1].set(a[0, _D:])
    h, s = _dense_part(entity_table, W, a8)
    h2 = jnp.concatenate([h[:, :_HD], h[:, _HD:]], axis=0)
    perm = []
    for g in range(_HD // 32):
        for i in range(16):
            perm.extend([g * 32 + i, g * 32 + 16 + i])
    h2b = h2.astype(jnp.bfloat16)[:, jnp.array(perm, jnp.int32)]
    s1p = jnp.pad(s[:, 0], (0, _NP - _N))
    s2p = jnp.pad(s[:, 1], (0, _NP - _N))

    e = edge_index.shape[1]
    pad = _NS * _NCH * _CH - e
    src3 = jnp.concatenate(
        [edge_index[0], jnp.full((pad,), _PAD_SRC, jnp.int32)]
    ).reshape(_NS, _NCH, _CH)
    dstp = jnp.concatenate(
        [edge_index[1], jnp.zeros((pad,), jnp.int32)])
    dst4 = jnp.stack([dstp, dstp + _N]).reshape(2, _NS, _NCH, _CH)

    hp = _sparse_part(h2b, s1p, s2p, src3, dst4)
    return jnp.concatenate([hp[0, :_N], hp[1, :_N]], axis=1)
